# baseline (speedup 1.0000x reference)
import functools

import jax
import jax.numpy as jnp
from jax import lax
from jax.experimental import pallas as pl
from jax.experimental.pallas import tpu as pltpu
from jax.experimental.pallas import tpu_sc as plsc


@functools.lru_cache(maxsize=None)
def _make_sc_kernel(N, D, E):
    info = plsc.get_sparse_core_info()
    NC, NS, L = info.num_cores, info.num_subcores, info.num_lanes
    NW = NC * NS
    C = 80
    R = 4
    per_w = E // NW
    assert E % NW == 0 and per_w % C == 0 and C % L == 0 and C <= 128
    n_chunks = per_w // C
    assert n_chunks % 2 == 1
    G = C // L

    mesh = plsc.VectorSubcoreMesh(core_axis_name="c", subcore_axis_name="s")

    idx_ring = [pltpu.VMEM((C,), jnp.int32) for _ in range(2 * R)]
    val_ring = [pltpu.VMEM((C,), jnp.float32) for _ in range(R)]

    @functools.partial(
        pl.kernel,
        mesh=mesh,
        out_type=jax.ShapeDtypeStruct((NW, L), jnp.float32),
        scratch_types=[
            pltpu.VMEM((C, D), jnp.float32),
            pltpu.VMEM((C, D), jnp.float32),
            pltpu.VMEM((C, D), jnp.float32),
            pltpu.VMEM((C, D), jnp.float32),
            pltpu.VMEM((L,), jnp.float32),
            pltpu.VMEM_SHARED((N, D), jnp.float32),
            *idx_ring,
            *val_ring,
            pltpu.SemaphoreType.DMA,
            pltpu.SemaphoreType.DMA,
            *[pltpu.SemaphoreType.DMA for _ in range(R)],
        ],
    )
    def k(x_hbm, rows_hbm, cols_hbm, vals_hbm, out_hbm,
          xr0, xr1, xc0, xc1, outv, xsh, *rest):
        rr = rest[0:R]
        cr = rest[R:2 * R]
        vr = rest[2 * R:3 * R]
        gsems = rest[3 * R:3 * R + 2]
        isems = rest[3 * R + 2:]
        wid = lax.axis_index("s") * NC + lax.axis_index("c")
        sid = lax.axis_index("s")
        base = wid * per_w
        xrs, xcs = (xr0, xr1), (xc0, xc1)

        del xsh, sid

        def idx_copies(chunk, slot):
            s = pl.ds(base + chunk * C, C)
            return (
                pltpu.make_async_copy(rows_hbm.at[s], rr[slot], isems[slot]),
                pltpu.make_async_copy(cols_hbm.at[s], cr[slot], isems[slot]),
                pltpu.make_async_copy(vals_hbm.at[s], vr[slot], isems[slot]),
            )

        def fire_idx(chunk, slot):
            for cp in idx_copies(chunk, slot):
                cp.start()

        def drain_idx(chunk, slot):
            for cp in idx_copies(chunk, slot):
                cp.wait()

        def gat_copies(rslot, gslot):
            return (
                pltpu.make_async_copy(x_hbm.at[rr[rslot]], xrs[gslot],
                                      gsems[gslot]),
                pltpu.make_async_copy(x_hbm.at[cr[rslot]], xcs[gslot],
                                      gsems[gslot]),
            )

        def fire_gather(rslot, gslot):
            for cp in gat_copies(rslot, gslot):
                cp.start()

        def drain_gather(rslot, gslot):
            for cp in gat_copies(rslot, gslot):
                cp.wait()

        def compute(gslot, rslot, acc):
            xr, xc = xrs[gslot], xcs[gslot]
            vb = vr[rslot]

            def group_body(g, acc):
                vv = vb[pl.ds(pl.multiple_of(g * L, L), L)]
                for u in range(L):
                    e = g * L + u
                    ve = vv[u]
                    for j in range(D // L):
                        a = xr[e, pl.ds(j * L, L)]
                        b = xc[e, pl.ds(j * L, L)]
                        acc = acc + ve * (a * b)
                return acc

            return lax.fori_loop(0, G, group_body, acc)

        for c in range(R):
            fire_idx(c, c % R)
        drain_idx(0, 0)
        fire_gather(0, 0)
        drain_idx(1, 1)
        fire_gather(1, 1)

        def half(c, par, acc):
            rslot, gslot = par % R, par % 2
            drain_gather(rslot, gslot)
            acc = compute(gslot, rslot, acc)
            @pl.when(c + R < n_chunks)
            def _():
                fire_idx(c + R, par % R)
            @pl.when(c + 2 < n_chunks)
            def _():
                drain_idx(c + 2, (par + 2) % R)
                fire_gather((par + 2) % R, gslot)
            return acc

        def quad_body(t, acc):
            c0 = 4 * t
            for p in range(4):
                acc = half(c0 + p, p, acc)
            return acc

        n_main = (n_chunks // 4) * 4
        acc = lax.fori_loop(0, n_chunks // 4, quad_body,
                            jnp.zeros((L,), jnp.float32))
        for p in range(n_chunks - n_main):
            acc = half(n_main + p, p, acc)

        outv[...] = acc
        pltpu.sync_copy(outv, out_hbm.at[wid])

    return k


def kernel(x, rows, cols, vals):
    N, D = x.shape
    E = vals.shape[0]
    k = _make_sc_kernel(N, D, E)
    out = k(x, rows.astype(jnp.int32), cols.astype(jnp.int32),
            vals.astype(jnp.float32))
    return jnp.sum(out) / E


# speedup vs baseline: 1.5559x; 1.5559x over previous
import functools

import jax
import jax.numpy as jnp
from jax import lax
from jax.experimental import pallas as pl
from jax.experimental.pallas import tpu as pltpu
from jax.experimental.pallas import tpu_sc as plsc


@functools.lru_cache(maxsize=None)
def _make_sc_kernel(N, D, E):
    info = plsc.get_sparse_core_info()
    NC, NS, L = info.num_cores, info.num_subcores, info.num_lanes
    NW = NC * NS
    C = 80
    R = 4
    per_w = E // NW
    assert E % NW == 0 and per_w % C == 0 and C % L == 0 and C <= 128
    n_chunks = per_w // C
    G = C // L

    ZR = ((N + NS - 1) // NS + C - 1) // C * C
    last_blocks, rem = divmod(N - (NS - 1) * ZR, C)
    assert rem == 0 and ZR % C == 0 and N > (NS - 1) * ZR

    mesh = plsc.VectorSubcoreMesh(core_axis_name="c", subcore_axis_name="s")

    @functools.partial(
        pl.kernel,
        mesh=mesh,
        out_type=jax.ShapeDtypeStruct((NW, L), jnp.float32),
        scratch_types=[
            *[pltpu.VMEM((C, D), jnp.float32) for _ in range(R)],
            *[pltpu.VMEM((C,), jnp.int32) for _ in range(R)],
            *[pltpu.VMEM((C,), jnp.int32) for _ in range(R)],
            *[pltpu.VMEM((C,), jnp.float32) for _ in range(R)],
            pltpu.VMEM((L,), jnp.float32),
            pltpu.VMEM_SHARED((N, D), jnp.float32),
            *[pltpu.SemaphoreType.DMA for _ in range(4 * R)],
        ],
    )
    def k(x_hbm, rows_hbm, cols_hbm, vals_hbm, out_hbm, *rest):
        gb = rest[0:R]
        cr = rest[R:2 * R]
        rr = rest[2 * R:3 * R]
        vr = rest[3 * R:4 * R]
        outv = rest[4 * R]
        lx = rest[4 * R + 1]
        ics = rest[4 * R + 2:5 * R + 2]
        irs = rest[5 * R + 2:6 * R + 2]
        gs = rest[6 * R + 2:7 * R + 2]
        ss = rest[7 * R + 2:8 * R + 2]
        wid = lax.axis_index("s") * NC + lax.axis_index("c")
        sid = lax.axis_index("s")
        base = wid * per_w

        def zfill(i, _):
            for j in range(D // L):
                gb[0][i, pl.ds(j * L, L)] = jnp.zeros((L,), jnp.float32)
            return 0
        lax.fori_loop(0, C, zfill, 0)
        row0 = sid * ZR

        def zcopy(b, _):
            r = pl.multiple_of(row0 + b * C, C)
            pltpu.sync_copy(gb[0], lx.at[pl.ds(r, C)])
            return 0
        nblk = jnp.where(sid == NS - 1, last_blocks, ZR // C)
        lax.fori_loop(0, nblk, zcopy, 0)
        plsc.subcore_barrier()

        def cv_copies(chunk, p):
            s = pl.ds(base + chunk * C, C)
            return (
                pltpu.make_async_copy(cols_hbm.at[s], cr[p], ics[p]),
                pltpu.make_async_copy(vals_hbm.at[s], vr[p], ics[p]),
            )

        def r_copies(chunk, p):
            s = pl.ds(base + chunk * C, C)
            return (pltpu.make_async_copy(rows_hbm.at[s], rr[p], irs[p]),)

        def g_copies(p):
            return (pltpu.make_async_copy(x_hbm.at[cr[p]], gb[p], gs[p]),)

        def s_copies(p):
            return (pltpu.make_async_copy(gb[p], lx.at[rr[p]], ss[p]),)

        def fire(copies, **kw):
            for cp in copies:
                cp.start(**kw)

        def drain(copies):
            for cp in copies:
                cp.wait()

        def scale(p):
            def group_body(g, _):
                vv = vr[p][pl.ds(pl.multiple_of(g * L, L), L)]
                for u in range(L):
                    e = g * L + u
                    ve = vv[u]
                    for j in range(D // L):
                        s = pl.ds(j * L, L)
                        gb[p][e, s] = ve * gb[p][e, s]
                return 0
            lax.fori_loop(0, G, group_body, 0)

        for c in range(R):
            fire(cv_copies(c, c))
        fire(r_copies(0, 0))
        fire(r_copies(1, 1))
        drain(cv_copies(0, 0))
        fire(g_copies(0))
        drain(cv_copies(1, 1))
        fire(g_copies(1))

        def half(c, p):
            @pl.when(c >= 2)
            def _():
                drain(s_copies((p + 2) % R))
            drain(g_copies(p))
            scale(p)
            drain(r_copies(c, p))
            fire(s_copies(p), add=True)
            @pl.when(c + R < n_chunks)
            def _():
                fire(cv_copies(c + R, p))
            @pl.when(c + 2 < n_chunks)
            def _():
                fire(r_copies(c + 2, (p + 2) % R))
                drain(cv_copies(c + 2, (p + 2) % R))
                fire(g_copies((p + 2) % R))

        def quad_body(t, _):
            for p in range(4):
                half(4 * t + p, p)
            return 0

        n_main = (n_chunks // 4) * 4
        lax.fori_loop(0, n_chunks // 4, quad_body, 0)
        for p in range(n_chunks - n_main):
            half(n_main + p, p)
        drain(s_copies((n_chunks - 2) % R))
        drain(s_copies((n_chunks - 1) % R))
        plsc.subcore_barrier()

        def dot_block(b, acc):
            r = pl.multiple_of(row0 + b * C, C)
            pltpu.sync_copy(x_hbm.at[pl.ds(r, C)], gb[0])
            pltpu.sync_copy(lx.at[pl.ds(r, C)], gb[1])

            def row_body(e, acc):
                for j in range(D // L):
                    s = pl.ds(j * L, L)
                    acc = acc + gb[0][e, s] * gb[1][e, s]
                return acc

            return lax.fori_loop(0, C, row_body, acc)

        acc = lax.fori_loop(0, nblk, dot_block,
                            jnp.zeros((L,), jnp.float32))
        outv[...] = acc
        pltpu.sync_copy(outv, out_hbm.at[wid])

    return k


def kernel(x, rows, cols, vals):
    N, D = x.shape
    E = vals.shape[0]
    k = _make_sc_kernel(N, D, E)
    out = k(x, rows.astype(jnp.int32), cols.astype(jnp.int32),
            vals.astype(jnp.float32))
    return jnp.sum(out) / E


# speedup vs baseline: 1.6373x; 1.0523x over previous
import functools

import jax
import jax.numpy as jnp
from jax import lax
from jax.experimental import pallas as pl
from jax.experimental.pallas import tpu as pltpu
from jax.experimental.pallas import tpu_sc as plsc


@functools.lru_cache(maxsize=None)
def _make_sc_kernel(N, D, E):
    info = plsc.get_sparse_core_info()
    NC, NS, L = info.num_cores, info.num_subcores, info.num_lanes
    NW = NC * NS
    C = 80
    R = 4
    per_w = E // NW
    assert E % NW == 0 and per_w % C == 0 and C % L == 0 and C <= 128
    n_chunks = per_w // C
    G = C // L

    ZR = ((N + NS - 1) // NS + C - 1) // C * C
    last_blocks, rem = divmod(N - (NS - 1) * ZR, C)
    assert rem == 0 and ZR % C == 0 and N > (NS - 1) * ZR

    mesh = plsc.VectorSubcoreMesh(core_axis_name="c", subcore_axis_name="s")

    @functools.partial(
        pl.kernel,
        mesh=mesh,
        out_type=jax.ShapeDtypeStruct((NW, L), jnp.float32),
        scratch_types=[
            *[pltpu.VMEM((C, D), jnp.float32) for _ in range(R)],
            *[pltpu.VMEM((C,), jnp.int32) for _ in range(R)],
            *[pltpu.VMEM((C,), jnp.int32) for _ in range(R)],
            *[pltpu.VMEM((C,), jnp.float32) for _ in range(R)],
            pltpu.VMEM((L,), jnp.float32),
            pltpu.VMEM_SHARED((N, D), jnp.float32),
            *[pltpu.SemaphoreType.DMA for _ in range(4 * R)],
        ],
    )
    def k(x_hbm, rows_hbm, cols_hbm, vals_hbm, out_hbm, *rest):
        gb = rest[0:R]
        cr = rest[R:2 * R]
        rr = rest[2 * R:3 * R]
        vr = rest[3 * R:4 * R]
        outv = rest[4 * R]
        lx = rest[4 * R + 1]
        ics = rest[4 * R + 2:5 * R + 2]
        irs = rest[5 * R + 2:6 * R + 2]
        gs = rest[6 * R + 2:7 * R + 2]
        ss = rest[7 * R + 2:8 * R + 2]
        wid = lax.axis_index("s") * NC + lax.axis_index("c")
        sid = lax.axis_index("s")
        base = wid * per_w

        def zfill(i, _):
            for j in range(D // L):
                gb[0][i, pl.ds(j * L, L)] = jnp.zeros((L,), jnp.float32)
            return 0
        lax.fori_loop(0, C, zfill, 0)
        row0 = sid * ZR

        def zcopy(b, _):
            r = pl.multiple_of(row0 + b * C, C)
            pltpu.sync_copy(gb[0], lx.at[pl.ds(r, C)])
            return 0
        nblk = jnp.where(sid == NS - 1, last_blocks, ZR // C)
        lax.fori_loop(0, nblk, zcopy, 0)
        plsc.subcore_barrier()

        def cv_copies(chunk, p):
            s = pl.ds(base + chunk * C, C)
            return (
                pltpu.make_async_copy(cols_hbm.at[s], cr[p], ics[p]),
                pltpu.make_async_copy(vals_hbm.at[s], vr[p], ics[p]),
            )

        def r_copies(chunk, p):
            s = pl.ds(base + chunk * C, C)
            return (pltpu.make_async_copy(rows_hbm.at[s], rr[p], irs[p]),)

        def g_copies(p):
            return (pltpu.make_async_copy(x_hbm.at[cr[p]], gb[p], gs[p]),)

        def s_copies(p):
            return (pltpu.make_async_copy(gb[p], lx.at[rr[p]], ss[p]),)

        def fire(copies, **kw):
            for cp in copies:
                cp.start(**kw)

        def drain(copies):
            for cp in copies:
                cp.wait()

        def scale(p):
            def group_body(g, _):
                vv = vr[p][pl.ds(pl.multiple_of(g * L, L), L)]
                for u in range(L):
                    e = g * L + u
                    ve = vv[u]
                    for j in range(D // L):
                        s = pl.ds(j * L, L)
                        gb[p][e, s] = ve * gb[p][e, s]
                return 0
            lax.fori_loop(0, G, group_body, 0)

        for c in range(R):
            fire(cv_copies(c, c))
        fire(r_copies(0, 0))
        fire(r_copies(1, 1))
        drain(cv_copies(0, 0))
        fire(g_copies(0))
        drain(cv_copies(1, 1))
        fire(g_copies(1))

        def half(c, p):
            @pl.when(c >= 2)
            def _():
                drain(s_copies((p + 2) % R))
            drain(g_copies(p))
            scale(p)
            drain(r_copies(c, p))
            fire(s_copies(p), add=True)
            @pl.when(c + R < n_chunks)
            def _():
                fire(cv_copies(c + R, p))
            @pl.when(c + 2 < n_chunks)
            def _():
                fire(r_copies(c + 2, (p + 2) % R))
                drain(cv_copies(c + 2, (p + 2) % R))
                fire(g_copies((p + 2) % R))

        def quad_body(t, _):
            for p in range(4):
                half(4 * t + p, p)
            return 0

        n_main = (n_chunks // 4) * 4
        lax.fori_loop(0, n_chunks // 4, quad_body, 0)
        for p in range(n_chunks - n_main):
            half(n_main + p, p)
        drain(s_copies((n_chunks - 2) % R))
        drain(s_copies((n_chunks - 1) % R))
        plsc.subcore_barrier()

        def xblk_copies(b, s):
            r = pl.multiple_of(row0 + b * C, C)
            return (pltpu.make_async_copy(x_hbm.at[pl.ds(r, C)], gb[2 * s],
                                          gs[s]),)

        def lblk_copies(b, s):
            r = pl.multiple_of(row0 + b * C, C)
            return (pltpu.make_async_copy(lx.at[pl.ds(r, C)], gb[2 * s + 1],
                                          ss[s]),)

        fire(xblk_copies(0, 0))
        fire(lblk_copies(0, 0))
        acc = jnp.zeros((L,), jnp.float32)
        maxb = ZR // C
        for b in range(maxb):
            s = b % 2
            @pl.when(b < nblk)
            def _():
                drain(xblk_copies(b, s))
                drain(lblk_copies(b, s))
            if b + 1 < maxb:
                @pl.when(b + 1 < nblk)
                def _():
                    fire(xblk_copies(b + 1, (b + 1) % 2))
                    fire(lblk_copies(b + 1, (b + 1) % 2))

            def row_body(e, a):
                for j in range(D // L):
                    sl = pl.ds(j * L, L)
                    a = a + gb[2 * s][e, sl] * gb[2 * s + 1][e, sl]
                return a

            blk_acc = lax.fori_loop(0, C, row_body,
                                    jnp.zeros((L,), jnp.float32))
            acc = acc + jnp.where(b < nblk, blk_acc, 0.0)
        outv[...] = acc
        pltpu.sync_copy(outv, out_hbm.at[wid])

    return k


def kernel(x, rows, cols, vals):
    N, D = x.shape
    E = vals.shape[0]
    k = _make_sc_kernel(N, D, E)
    out = k(x, rows.astype(jnp.int32), cols.astype(jnp.int32),
            vals.astype(jnp.float32))
    return jnp.sum(out) / E


# speedup vs baseline: 1.6502x; 1.0079x over previous
import functools

import jax
import jax.numpy as jnp
from jax import lax
from jax.experimental import pallas as pl
from jax.experimental.pallas import tpu as pltpu
from jax.experimental.pallas import tpu_sc as plsc


@functools.lru_cache(maxsize=None)
def _make_sc_kernel(N, D, E):
    info = plsc.get_sparse_core_info()
    NC, NS, L = info.num_cores, info.num_subcores, info.num_lanes
    NW = NC * NS
    C = 80
    R = 4
    per_w = E // NW
    assert E % NW == 0 and per_w % C == 0 and C % L == 0 and C <= 128
    n_chunks = per_w // C
    G = C // L

    ZR = ((N + NS - 1) // NS + C - 1) // C * C
    last_blocks, rem = divmod(N - (NS - 1) * ZR, C)
    assert rem == 0 and ZR % C == 0 and N > (NS - 1) * ZR

    mesh = plsc.VectorSubcoreMesh(core_axis_name="c", subcore_axis_name="s")

    @functools.partial(
        pl.kernel,
        mesh=mesh,
        out_type=jax.ShapeDtypeStruct((NW, L), jnp.float32),
        scratch_types=[
            *[pltpu.VMEM((C, D), jnp.float32) for _ in range(R)],
            *[pltpu.VMEM((C,), jnp.int32) for _ in range(R)],
            *[pltpu.VMEM((C,), jnp.int32) for _ in range(R)],
            *[pltpu.VMEM((C,), jnp.float32) for _ in range(R)],
            pltpu.VMEM((L,), jnp.float32),
            pltpu.VMEM_SHARED((N, D), jnp.float32),
            *[pltpu.SemaphoreType.DMA for _ in range(4 * R)],
        ],
    )
    def k(x_hbm, rows_hbm, cols_hbm, vals_hbm, out_hbm, *rest):
        gb = rest[0:R]
        cr = rest[R:2 * R]
        rr = rest[2 * R:3 * R]
        vr = rest[3 * R:4 * R]
        outv = rest[4 * R]
        lx = rest[4 * R + 1]
        ics = rest[4 * R + 2:5 * R + 2]
        irs = rest[5 * R + 2:6 * R + 2]
        gs = rest[6 * R + 2:7 * R + 2]
        ss = rest[7 * R + 2:8 * R + 2]
        wid = lax.axis_index("s") * NC + lax.axis_index("c")
        sid = lax.axis_index("s")
        base = wid * per_w

        def zfill(i, _):
            for j in range(D // L):
                gb[2][i, pl.ds(j * L, L)] = jnp.zeros((L,), jnp.float32)
            return 0
        lax.fori_loop(0, C, zfill, 0)
        row0 = sid * ZR
        nblk = jnp.where(sid == NS - 1, last_blocks, ZR // C)

        def cv_copies(chunk, p):
            s = pl.ds(base + chunk * C, C)
            return (
                pltpu.make_async_copy(cols_hbm.at[s], cr[p], ics[p]),
                pltpu.make_async_copy(vals_hbm.at[s], vr[p], ics[p]),
            )

        def r_copies(chunk, p):
            s = pl.ds(base + chunk * C, C)
            return (pltpu.make_async_copy(rows_hbm.at[s], rr[p], irs[p]),)

        def g_copies(p):
            return (pltpu.make_async_copy(x_hbm.at[cr[p]], gb[p], gs[p]),)

        def s_copies(p):
            return (pltpu.make_async_copy(gb[p], lx.at[rr[p]], ss[p]),)

        def fire(copies, **kw):
            for cp in copies:
                cp.start(**kw)

        def drain(copies):
            for cp in copies:
                cp.wait()

        def scale(p):
            def group_body(g, _):
                vv = vr[p][pl.ds(pl.multiple_of(g * L, L), L)]
                for u in range(L):
                    e = g * L + u
                    ve = vv[u]
                    for j in range(D // L):
                        s = pl.ds(j * L, L)
                        gb[p][e, s] = ve * gb[p][e, s]
                return 0
            lax.fori_loop(0, G, group_body, 0)

        for c in range(R):
            fire(cv_copies(c, c))
        fire(r_copies(0, 0))
        fire(r_copies(1, 1))
        drain(cv_copies(0, 0))
        fire(g_copies(0))
        drain(cv_copies(1, 1))
        fire(g_copies(1))

        def zcopy(b, _):
            r = pl.multiple_of(row0 + b * C, C)
            pltpu.sync_copy(gb[2], lx.at[pl.ds(r, C)])
            return 0
        lax.fori_loop(0, nblk, zcopy, 0)
        plsc.subcore_barrier()

        def half(c, p):
            @pl.when(c >= 2)
            def _():
                drain(s_copies((p + 2) % R))
            drain(g_copies(p))
            scale(p)
            drain(r_copies(c, p))
            fire(s_copies(p), add=True)
            @pl.when(c + R < n_chunks)
            def _():
                fire(cv_copies(c + R, p))
            @pl.when(c + 2 < n_chunks)
            def _():
                fire(r_copies(c + 2, (p + 2) % R))
                drain(cv_copies(c + 2, (p + 2) % R))
                fire(g_copies((p + 2) % R))

        def quad_body(t, _):
            for p in range(4):
                half(4 * t + p, p)
            return 0

        n_main = (n_chunks // 4) * 4
        lax.fori_loop(0, n_chunks // 4, quad_body, 0)
        for p in range(n_chunks - n_main):
            half(n_main + p, p)
        drain(s_copies((n_chunks - 2) % R))
        drain(s_copies((n_chunks - 1) % R))
        plsc.subcore_barrier()

        def xblk_copies(b, s):
            r = pl.multiple_of(row0 + b * C, C)
            return (pltpu.make_async_copy(x_hbm.at[pl.ds(r, C)], gb[2 * s],
                                          gs[s]),)

        def lblk_copies(b, s):
            r = pl.multiple_of(row0 + b * C, C)
            return (pltpu.make_async_copy(lx.at[pl.ds(r, C)], gb[2 * s + 1],
                                          ss[s]),)

        fire(xblk_copies(0, 0))
        fire(lblk_copies(0, 0))
        acc = jnp.zeros((L,), jnp.float32)
        maxb = ZR // C
        for b in range(maxb):
            s = b % 2
            @pl.when(b < nblk)
            def _():
                drain(xblk_copies(b, s))
                drain(lblk_copies(b, s))
            if b + 1 < maxb:
                @pl.when(b + 1 < nblk)
                def _():
                    fire(xblk_copies(b + 1, (b + 1) % 2))
                    fire(lblk_copies(b + 1, (b + 1) % 2))

            def row_body(e, a):
                for j in range(D // L):
                    sl = pl.ds(j * L, L)
                    a = a + gb[2 * s][e, sl] * gb[2 * s + 1][e, sl]
                return a

            blk_acc = lax.fori_loop(0, C, row_body,
                                    jnp.zeros((L,), jnp.float32))
            acc = acc + jnp.where(b < nblk, blk_acc, 0.0)
        outv[...] = acc
        pltpu.sync_copy(outv, out_hbm.at[wid])

    return k


def kernel(x, rows, cols, vals):
    N, D = x.shape
    E = vals.shape[0]
    k = _make_sc_kernel(N, D, E)
    out = k(x, rows.astype(jnp.int32), cols.astype(jnp.int32),
            vals.astype(jnp.float32))
    return jnp.sum(out) / E
